# baseline (speedup 1.0000x reference)
import functools

import jax
import jax.numpy as jnp
from jax import lax
from jax.experimental import pallas as pl
from jax.experimental.pallas import tpu as pltpu
from jax.experimental.pallas import tpu_sc as plsc

VOCAB = 49408
HIDDEN = 512
MAX_POS = 77
BATCH = 4096
SEQ = 77

LANES = 16
NUM_CORES = 2
NUM_SUBCORES = 16
NUM_WORKERS = NUM_CORES * NUM_SUBCORES
BPW = BATCH // NUM_WORKERS
CCHUNKS = HIDDEN // LANES
SEQ_PAD = 80

_mesh = plsc.VectorSubcoreMesh(core_axis_name="c", subcore_axis_name="s")


@functools.partial(
    pl.kernel,
    mesh=_mesh,
    out_type=jax.ShapeDtypeStruct((BATCH, SEQ_PAD, HIDDEN), jnp.float32),
    scratch_types=[
        pltpu.VMEM((SEQ_PAD,), jnp.int32),
        pltpu.VMEM((SEQ_PAD,), jnp.int32),
        pltpu.VMEM((SEQ_PAD, HIDDEN), jnp.float32),
        pltpu.VMEM((SEQ_PAD, HIDDEN), jnp.float32),
        pltpu.VMEM((SEQ_PAD, HIDDEN), jnp.float32),
        pltpu.SemaphoreType.DMA,
        pltpu.SemaphoreType.DMA,
        pltpu.SemaphoreType.DMA,
        pltpu.SemaphoreType.DMA,
        pltpu.SemaphoreType.DMA,
        pltpu.SemaphoreType.DMA,
        pltpu.SemaphoreType.DMA,
    ],
)
def _emb_kernel(ids_hbm, tok_hbm, pos_hbm, out_hbm,
                irow0, irow1, pos_v, buf_a, buf_b,
                isem0, isem1, gsem_a, gsem_b, ssem_a, ssem_b, psem):
    wid = lax.axis_index("s") * NUM_CORES + lax.axis_index("c")
    bbase = wid * BPW

    pltpu.async_copy(pos_hbm, pos_v, psem).wait()

    irows = ((irow0, isem0), (irow1, isem1))
    slots = ((buf_a, gsem_a, ssem_a), (buf_b, gsem_b, ssem_b))

    def idx_src(g):
        return ids_hbm.at[pl.ds((bbase + g) * SEQ_PAD, SEQ_PAD)]

    def start_idx(g, ir):
        pltpu.async_copy(idx_src(g), ir[0], ir[1])

    def wait_idx(g, ir):
        pltpu.make_async_copy(idx_src(g), ir[0], ir[1]).wait()

    def start_gather(slot, ir):
        pltpu.async_copy(tok_hbm.at[ir[0]], slot[0], slot[1])

    def wait_gather(slot, ir):
        pltpu.make_async_copy(tok_hbm.at[ir[0]], slot[0], slot[1]).wait()

    start_idx(0, irows[0])
    wait_idx(0, irows[0])
    start_gather(slots[0], irows[0])
    start_idx(1, irows[1])

    def pair(gg, carry):
        for b in range(2):
            g = gg * 2 + b
            cur, oth = slots[b], slots[1 - b]
            irc, irn = irows[b], irows[1 - b]

            wait_gather(cur, irc)

            @pl.when(g >= 1)
            def _():
                pltpu.make_async_copy(oth[0], out_hbm.at[bbase + g - 1],
                                      oth[2]).wait()

            @pl.when(g + 2 < BPW)
            def _():
                start_idx(g + 2, irc)

            @pl.when(g + 1 < BPW)
            def _():
                wait_idx(g + 1, irn)
                start_gather(oth, irn)

            @plsc.parallel_loop(0, SEQ_PAD, 1)
            def _(r):
                for c in range(CCHUNKS):
                    sl = pl.ds(c * LANES, LANES)
                    cur[0][r, sl] = cur[0][r, sl] + pos_v[r, sl]

            pltpu.async_copy(cur[0], out_hbm.at[bbase + g], cur[2])
        return carry

    lax.fori_loop(0, BPW // 2, pair, 0)
    pltpu.make_async_copy(slots[1][0], out_hbm.at[bbase + BPW - 1],
                          slots[1][2]).wait()


def kernel(input_ids, token_embedding, position_embedding):
    ids = jnp.pad(input_ids.astype(jnp.int32), ((0, 0), (0, SEQ_PAD - SEQ)))
    pos = jnp.pad(position_embedding, ((0, SEQ_PAD - MAX_POS), (0, 0)))
    out = _emb_kernel(ids.reshape(-1), token_embedding, pos)
    return out[:, :SEQ, :]


# speedup vs baseline: 1.1570x; 1.1570x over previous
import functools

import jax
import jax.numpy as jnp
from jax import lax
from jax.experimental import pallas as pl
from jax.experimental.pallas import tpu as pltpu
from jax.experimental.pallas import tpu_sc as plsc

VOCAB = 49408
HIDDEN = 512
MAX_POS = 77
BATCH = 4096
SEQ = 77

LANES = 16
NUM_CORES = 2
NUM_SUBCORES = 16
NUM_WORKERS = NUM_CORES * NUM_SUBCORES
ROWS = BATCH * SEQ
RPW = ROWS // NUM_WORKERS
CHUNK = 64
CPW = RPW // CHUNK
CCHUNKS = HIDDEN // LANES

_mesh = plsc.VectorSubcoreMesh(core_axis_name="c", subcore_axis_name="s")


@functools.partial(
    pl.kernel,
    mesh=_mesh,
    out_type=jax.ShapeDtypeStruct((ROWS // CHUNK, CHUNK, HIDDEN), jnp.float32),
    scratch_types=[
        pltpu.VMEM((CPW, CHUNK), jnp.int32),
        pltpu.VMEM((SEQ, HIDDEN), jnp.float32),
        pltpu.VMEM((CHUNK, HIDDEN), jnp.float32),
        pltpu.VMEM((CHUNK, HIDDEN), jnp.float32),
        pltpu.SemaphoreType.DMA,
        pltpu.SemaphoreType.DMA,
        pltpu.SemaphoreType.DMA,
        pltpu.SemaphoreType.DMA,
        pltpu.SemaphoreType.DMA,
    ],
)
def _emb_kernel(ids_hbm, tok_hbm, pos_hbm, out_hbm,
                idx_v, pos_v, buf_a, buf_b,
                gsem_a, gsem_b, ssem_a, ssem_b, psem):
    wid = lax.axis_index("s") * NUM_CORES + lax.axis_index("c")
    out_base = wid * CPW

    pltpu.sync_copy(ids_hbm.at[wid], idx_v)
    pltpu.async_copy(pos_hbm, pos_v, psem).wait()

    slots = ((buf_a, gsem_a, ssem_a), (buf_b, gsem_b, ssem_b))

    def start_gather(c, slot):
        pltpu.async_copy(tok_hbm.at[idx_v.at[c]], slot[0], slot[1])

    def wait_gather(c, slot):
        pltpu.make_async_copy(tok_hbm.at[idx_v.at[c]], slot[0], slot[1]).wait()

    def start_scatter(c, slot):
        pltpu.async_copy(slot[0], out_hbm.at[out_base + c], slot[2])

    def wait_scatter(c, slot):
        pltpu.make_async_copy(slot[0], out_hbm.at[out_base + c], slot[2]).wait()

    start_gather(0, slots[0])

    def pair(cc, carry):
        for b in range(2):
            c = cc * 2 + b
            cur = slots[b]
            oth = slots[1 - b]
            @pl.when(c >= 1)
            def _():
                wait_scatter(c - 1, oth)

            @pl.when(c + 1 < CPW)
            def _():
                start_gather(c + 1, oth)

            wait_gather(c, cur)

            phase = lax.rem(c * CHUNK, SEQ)

            @plsc.parallel_loop(0, CHUNK, 1)
            def _(i):
                s = phase + i
                s = jnp.where(s >= SEQ, s - SEQ, s)
                for k in range(CCHUNKS):
                    sl = pl.ds(k * LANES, LANES)
                    cur[0][i, sl] = cur[0][i, sl] + pos_v[s, sl]

            start_scatter(c, cur)
        return carry

    lax.fori_loop(0, CPW // 2, pair, 0)
    wait_scatter(CPW - 1, slots[1])


def kernel(input_ids, token_embedding, position_embedding):
    ids = input_ids.astype(jnp.int32).reshape(NUM_WORKERS, CPW, CHUNK)
    out = _emb_kernel(ids, token_embedding, position_embedding)
    return out.reshape(BATCH, SEQ, HIDDEN)
